# baseline (speedup 1.0000x reference)
import functools

import jax
import jax.numpy as jnp
from jax import lax
from jax.experimental import pallas as pl
from jax.experimental.pallas import tpu as pltpu
from jax.experimental.pallas import tpu_sc as plsc

_NT = 12048
_D = 256
_DH = 128
_NC = 2
_NS = 16
_L = 16
_G = 128
_GPC = 2
_CH = _G * _GPC
_RGRP = 8
_RPT = 752
_RB = 2008


def _codemap_body(f2_ref, wct_ref, bc_ref, o_ref):
    o_ref[...] = (
        jnp.dot(f2_ref[...], wct_ref[...], preferred_element_type=jnp.float32)
        + bc_ref[...]
    )


def _mlp_body(x_ref, w1t_ref, b1_ref, w2t_ref, b2_ref, o_ref):
    h = jnp.dot(x_ref[...], w1t_ref[...], preferred_element_type=jnp.float32)
    h = h + b1_ref[...]
    h = jnp.where(h > 0, h, 0.1 * h)
    o = jnp.dot(h, w2t_ref[...], preferred_element_type=jnp.float32)
    o = o + b2_ref[...]
    o_ref[0] = o[:, :_DH]
    o_ref[1] = o[:, _DH:]


def _sc_body(nchunk, h_hbm, rowi_hbm, coli_hbm, vali_hbm, out_hbm, y_hbm,
             rowv, colv, valv, rows_v, acc, isem, gsem, ssem):
    c = lax.axis_index("c")
    s = lax.axis_index("s")
    cbias = c * _NT
    ebase = s * (nchunk * _CH)
    ngrp = nchunk * _GPC
    zerov = jnp.zeros((_L,), jnp.float32)

    def _zero_acc():
        def zbody(r, carry):
            for fb in range(_DH // _L):
                rows_v[r, pl.ds(fb * _L, _L)] = zerov
            return carry
        lax.fori_loop(0, _CH, zbody, 0)
        for off in range(0, _RPT, _CH):
            n = min(_CH, _RPT - off)
            pltpu.sync_copy(rows_v.at[pl.ds(0, n)],
                            acc.at[pl.ds(s * _RPT + off, n)])

        @pl.when(s == _NS - 1)
        def _():
            pltpu.sync_copy(rows_v.at[pl.ds(0, _NT - _NS * _RPT)],
                            acc.at[pl.ds(_NS * _RPT, _NT - _NS * _RPT)])

    def _pass(table_hbm, dst_hbm):
        kper = _RGRP // _GPC

        def sca_desc(k, j):
            return pltpu.make_async_copy(
                rows_v.at[pl.ds(j * _G, _G)],
                acc.at[rowv.at[lax.rem(k, kper) * _GPC + j]], ssem)

        pltpu.async_copy(coli_hbm.at[pl.ds(ebase, _CH)],
                         colv.at[pl.ds(0, _CH)], isem)

        def chunk(k, carry):
            e0 = ebase + k * _CH
            co = lax.rem(k, 2) * _CH

            pltpu.make_async_copy(coli_hbm.at[pl.ds(e0, _CH)],
                                  colv.at[pl.ds(co, _CH)], isem).wait()

            def bias(t, cc):
                colv[pl.ds(co + t * _L, _L)] = (
                    colv[pl.ds(co + t * _L, _L)] + cbias)
                return cc
            lax.fori_loop(0, _CH // _L, bias, 0)

            @pl.when(k >= 1)
            def _():
                sca_desc(k - 1, 0).wait()
                sca_desc(k - 1, 1).wait()

            @pl.when(lax.rem(k, kper) == 0)
            def _():
                g0 = s * ngrp + (k // kper) * _RGRP
                pltpu.sync_copy(rowi_hbm.at[pl.ds(g0, _RGRP)], rowv)

            hg = _G // 2
            gat = [
                pltpu.async_copy(
                    table_hbm.at[colv.at[pl.ds(co + q * hg, hg)]],
                    rows_v.at[pl.ds(q * hg, hg)], gsem)
                for q in range(2 * _GPC)
            ]

            @pl.when(k < nchunk - 1)
            def _():
                pltpu.async_copy(
                    coli_hbm.at[pl.ds(e0 + _CH, _CH)],
                    colv.at[pl.ds(_CH - co, _CH)], isem)

            pltpu.sync_copy(vali_hbm.at[pl.ds(e0, _CH)], valv)

            for j in range(_GPC):
                gat[2 * j].wait()
                gat[2 * j + 1].wait()

                def scale(t, cc, j=j):
                    val16 = valv[pl.ds(j * _G + t * _L, _L)]
                    for u in range(_L):
                        r = j * _G + t * _L + u
                        v = val16[u]
                        for fb in range(_DH // _L):
                            rows_v[r, pl.ds(fb * _L, _L)] = (
                                rows_v[r, pl.ds(fb * _L, _L)] * v)
                    return cc
                lax.fori_loop(0, _G // _L, scale, 0)
                pltpu.async_copy(
                    rows_v.at[pl.ds(j * _G, _G)],
                    acc.at[rowv.at[lax.rem(k, kper) * _GPC + j]],
                    ssem, priority=1, add=True)
            return carry
        lax.fori_loop(0, nchunk, chunk, 0)
        sca_desc(nchunk - 1, 0).wait()
        sca_desc(nchunk - 1, 1).wait()
        plsc.subcore_barrier()
        pltpu.sync_copy(acc.at[pl.ds(s * _RPT, _RPT)],
                        dst_hbm.at[pl.ds(cbias + s * _RPT, _RPT)])

        @pl.when(s == _NS - 1)
        def _():
            rem = _NT - _NS * _RPT
            pltpu.sync_copy(acc.at[pl.ds(_NS * _RPT, rem)],
                            dst_hbm.at[pl.ds(cbias + _NS * _RPT, rem)])

    _zero_acc()
    plsc.subcore_barrier()
    _pass(h_hbm, y_hbm)
    _zero_acc()
    plsc.subcore_barrier()
    _pass(y_hbm, out_hbm)


def kernel(seq_a, adj_indices, adj_values, node_emb, init_fea2, Wc, bc,
           W1, b1, W2, b2):
    del seq_a

    cm = pl.pallas_call(
        _codemap_body,
        out_shape=jax.ShapeDtypeStruct((init_fea2.shape[0], _D), jnp.float32),
    )(init_fea2, Wc.T, bc[None, :])
    x = jnp.concatenate([node_emb, cm], axis=0)

    nblk = _NT // _RB
    h_split = pl.pallas_call(
        _mlp_body,
        grid=(nblk,),
        in_specs=[
            pl.BlockSpec((_RB, _D), lambda i: (i, 0)),
            pl.BlockSpec((_D, W1.shape[0]), lambda i: (0, 0)),
            pl.BlockSpec((1, W1.shape[0]), lambda i: (0, 0)),
            pl.BlockSpec((W1.shape[0], _D), lambda i: (0, 0)),
            pl.BlockSpec((1, _D), lambda i: (0, 0)),
        ],
        out_specs=pl.BlockSpec((_NC, _RB, _DH), lambda i: (0, i, 0)),
        out_shape=jax.ShapeDtypeStruct((_NC, _NT, _DH), jnp.float32),
    )(x, W1.T, b1[None, :], W2.T, b2[None, :])
    h2 = h_split.reshape(_NC * _NT, _DH)

    e = adj_values.shape[0]
    epad = -(-e // (_NS * _CH)) * (_NS * _CH)
    rows = adj_indices[0].astype(jnp.int32)
    cols = adj_indices[1].astype(jnp.int32)
    vals = adj_values
    if epad != e:
        pad = epad - e
        rows = jnp.concatenate([rows, jnp.zeros((pad,), jnp.int32)])
        cols = jnp.concatenate([cols, jnp.zeros((pad,), jnp.int32)])
        vals = jnp.concatenate([vals, jnp.zeros((pad,), jnp.float32)])
    rows2 = rows.reshape(-1, _G)
    nchunk = epad // (_NS * _CH)

    mesh = plsc.VectorSubcoreMesh(core_axis_name="c", subcore_axis_name="s",
                                  num_cores=_NC, num_subcores=_NS)
    sc = pl.kernel(
        functools.partial(_sc_body, nchunk),
        out_type=(
            jax.ShapeDtypeStruct((_NC * _NT, _DH), jnp.float32),
            jax.ShapeDtypeStruct((_NC * _NT, _DH), jnp.float32),
        ),
        mesh=mesh,
        scratch_types=[
            pltpu.VMEM((_RGRP, _G), jnp.int32),
            pltpu.VMEM((2 * _CH,), jnp.int32),
            pltpu.VMEM((_CH,), jnp.float32),
            pltpu.VMEM((_CH, _DH), jnp.float32),
            pltpu.VMEM_SHARED((_NT, _DH), jnp.float32),
            pltpu.SemaphoreType.DMA,
            pltpu.SemaphoreType.DMA,
            pltpu.SemaphoreType.DMA,
        ],
    )
    out2, _y = sc(h2, rows2, cols, vals)
    o = out2.reshape(_NC, _NT, _DH)
    return jnp.concatenate([o[0], o[1]], axis=1)


# speedup vs baseline: 1.0072x; 1.0072x over previous
import functools

import jax
import jax.numpy as jnp
from jax import lax
from jax.experimental import pallas as pl
from jax.experimental.pallas import tpu as pltpu
from jax.experimental.pallas import tpu_sc as plsc

_NT = 12048
_D = 256
_DH = 128
_NC = 2
_NS = 16
_L = 16
_G = 128
_GPC = 2
_CH = _G * _GPC
_RGRP = 8
_RPT = 752
_RB = 2008


def _codemap_body(f2_ref, wct_ref, bc_ref, o_ref):
    o_ref[...] = (
        jnp.dot(f2_ref[...], wct_ref[...], preferred_element_type=jnp.float32)
        + bc_ref[...]
    )


def _mlp_body(x_ref, w1t_ref, b1_ref, w2t_ref, b2_ref, o_ref):
    h = jnp.dot(x_ref[...], w1t_ref[...], preferred_element_type=jnp.float32)
    h = h + b1_ref[...]
    h = jnp.where(h > 0, h, 0.1 * h)
    o = jnp.dot(h, w2t_ref[...], preferred_element_type=jnp.float32)
    o = o + b2_ref[...]
    o_ref[0] = o[:, :_DH]
    o_ref[1] = o[:, _DH:]


def _sc_body(nchunk, h_hbm, rowi_hbm, coli_hbm, vali_hbm, out_hbm, y_hbm,
             rowv, colv, valv, rows_v, acc, isem, gsem, ssem):
    c = lax.axis_index("c")
    s = lax.axis_index("s")
    cbias = c * _NT
    ebase = s * (nchunk * _CH)
    ngrp = nchunk * _GPC
    zerov = jnp.zeros((_L,), jnp.float32)

    def _zero_acc():
        def zbody(r, carry):
            for fb in range(_DH // _L):
                rows_v[r, pl.ds(fb * _L, _L)] = zerov
            return carry
        lax.fori_loop(0, _CH, zbody, 0)
        for off in range(0, _RPT, _CH):
            n = min(_CH, _RPT - off)
            pltpu.sync_copy(rows_v.at[pl.ds(0, n)],
                            acc.at[pl.ds(s * _RPT + off, n)])

        @pl.when(s == _NS - 1)
        def _():
            pltpu.sync_copy(rows_v.at[pl.ds(0, _NT - _NS * _RPT)],
                            acc.at[pl.ds(_NS * _RPT, _NT - _NS * _RPT)])

    def _pass(table_hbm, dst_fn):
        kper = _RGRP // _GPC

        def sca_desc(k, j):
            return pltpu.make_async_copy(
                rows_v.at[pl.ds(j * _G, _G)],
                acc.at[rowv.at[lax.rem(k, kper) * _GPC + j]], ssem)

        pltpu.async_copy(coli_hbm.at[pl.ds(ebase, _CH)],
                         colv.at[pl.ds(0, _CH)], isem)

        def chunk(k, carry):
            e0 = ebase + k * _CH
            co = lax.rem(k, 2) * _CH

            pltpu.make_async_copy(coli_hbm.at[pl.ds(e0, _CH)],
                                  colv.at[pl.ds(co, _CH)], isem).wait()

            def bias(t, cc):
                colv[pl.ds(co + t * _L, _L)] = (
                    colv[pl.ds(co + t * _L, _L)] + cbias)
                return cc
            lax.fori_loop(0, _CH // _L, bias, 0)

            @pl.when(k >= 1)
            def _():
                sca_desc(k - 1, 0).wait()
                sca_desc(k - 1, 1).wait()

            @pl.when(lax.rem(k, kper) == 0)
            def _():
                g0 = s * ngrp + (k // kper) * _RGRP
                pltpu.sync_copy(rowi_hbm.at[pl.ds(g0, _RGRP)], rowv)

            gat = [
                pltpu.async_copy(
                    table_hbm.at[colv.at[pl.ds(co + j * _G, _G)]],
                    rows_v.at[pl.ds(j * _G, _G)], gsem)
                for j in range(_GPC)
            ]

            @pl.when(k < nchunk - 1)
            def _():
                pltpu.async_copy(
                    coli_hbm.at[pl.ds(e0 + _CH, _CH)],
                    colv.at[pl.ds(_CH - co, _CH)], isem)

            pltpu.sync_copy(vali_hbm.at[pl.ds(e0, _CH)], valv)

            for j in range(_GPC):
                gat[j].wait()

                def scale(t, cc, j=j):
                    val16 = valv[pl.ds(j * _G + t * _L, _L)]
                    for u in range(_L):
                        r = j * _G + t * _L + u
                        v = val16[u]
                        for fb in range(_DH // _L):
                            rows_v[r, pl.ds(fb * _L, _L)] = (
                                rows_v[r, pl.ds(fb * _L, _L)] * v)
                    return cc
                lax.fori_loop(0, _G // _L, scale, 0)
                pltpu.async_copy(
                    rows_v.at[pl.ds(j * _G, _G)],
                    acc.at[rowv.at[lax.rem(k, kper) * _GPC + j]],
                    ssem, add=True)
            return carry
        lax.fori_loop(0, nchunk, chunk, 0)
        sca_desc(nchunk - 1, 0).wait()
        sca_desc(nchunk - 1, 1).wait()
        plsc.subcore_barrier()
        dst_fn(s * _RPT, _RPT)

        @pl.when(s == _NS - 1)
        def _():
            dst_fn(_NS * _RPT, _NT - _NS * _RPT)

    def y_dst(r, n):
        pltpu.sync_copy(acc.at[pl.ds(r, n)],
                        y_hbm.at[pl.ds(cbias + r, n)])

    def out_dst(r, n):
        pltpu.sync_copy(acc.at[pl.ds(r, n)],
                        out_hbm.at[pl.ds(r, n), pl.ds(c * _DH, _DH)])

    _zero_acc()
    plsc.subcore_barrier()
    _pass(h_hbm, y_dst)
    _zero_acc()
    plsc.subcore_barrier()
    _pass(y_hbm, out_dst)


def kernel(seq_a, adj_indices, adj_values, node_emb, init_fea2, Wc, bc,
           W1, b1, W2, b2):
    del seq_a

    cm = pl.pallas_call(
        _codemap_body,
        out_shape=jax.ShapeDtypeStruct((init_fea2.shape[0], _D), jnp.float32),
    )(init_fea2, Wc.T, bc[None, :])
    x = jnp.concatenate([node_emb, cm], axis=0)

    nblk = _NT // _RB
    h_split = pl.pallas_call(
        _mlp_body,
        grid=(nblk,),
        in_specs=[
            pl.BlockSpec((_RB, _D), lambda i: (i, 0)),
            pl.BlockSpec((_D, W1.shape[0]), lambda i: (0, 0)),
            pl.BlockSpec((1, W1.shape[0]), lambda i: (0, 0)),
            pl.BlockSpec((W1.shape[0], _D), lambda i: (0, 0)),
            pl.BlockSpec((1, _D), lambda i: (0, 0)),
        ],
        out_specs=pl.BlockSpec((_NC, _RB, _DH), lambda i: (0, i, 0)),
        out_shape=jax.ShapeDtypeStruct((_NC, _NT, _DH), jnp.float32),
    )(x, W1.T, b1[None, :], W2.T, b2[None, :])
    h2 = h_split.reshape(_NC * _NT, _DH)

    e = adj_values.shape[0]
    epad = -(-e // (_NS * _CH)) * (_NS * _CH)
    rows = adj_indices[0].astype(jnp.int32)
    cols = adj_indices[1].astype(jnp.int32)
    vals = adj_values
    if epad != e:
        pad = epad - e
        rows = jnp.concatenate([rows, jnp.zeros((pad,), jnp.int32)])
        cols = jnp.concatenate([cols, jnp.zeros((pad,), jnp.int32)])
        vals = jnp.concatenate([vals, jnp.zeros((pad,), jnp.float32)])
    rows2 = rows.reshape(-1, _G)
    nchunk = epad // (_NS * _CH)

    mesh = plsc.VectorSubcoreMesh(core_axis_name="c", subcore_axis_name="s",
                                  num_cores=_NC, num_subcores=_NS)
    sc = pl.kernel(
        functools.partial(_sc_body, nchunk),
        out_type=(
            jax.ShapeDtypeStruct((_NT, _D), jnp.float32),
            jax.ShapeDtypeStruct((_NC * _NT, _DH), jnp.float32),
        ),
        mesh=mesh,
        scratch_types=[
            pltpu.VMEM((_RGRP, _G), jnp.int32),
            pltpu.VMEM((2 * _CH,), jnp.int32),
            pltpu.VMEM((_CH,), jnp.float32),
            pltpu.VMEM((_CH, _DH), jnp.float32),
            pltpu.VMEM_SHARED((_NT, _DH), jnp.float32),
            pltpu.SemaphoreType.DMA,
            pltpu.SemaphoreType.DMA,
            pltpu.SemaphoreType.DMA,
        ],
    )
    out, _y = sc(h2, rows2, cols, vals)
    return out


# speedup vs baseline: 1.1254x; 1.1174x over previous
import functools

import jax
import jax.numpy as jnp
from jax import lax
from jax.experimental import pallas as pl
from jax.experimental.pallas import tpu as pltpu
from jax.experimental.pallas import tpu_sc as plsc

_NT = 12048
_D = 256
_DH = 128
_NC = 2
_NS = 16
_L = 16
_G = 128
_GPC = 2
_CH = _G * _GPC
_RGRP = 8
_RPT = 752
_RB = 2008


def _codemap_body(f2_ref, wct_ref, bc_ref, o_ref):
    o_ref[...] = (
        jnp.dot(f2_ref[...], wct_ref[...], preferred_element_type=jnp.float32)
        + bc_ref[...]
    )


def _mlp_body(x_ref, w1t_ref, b1_ref, w2t_ref, b2_ref, o_ref):
    h = jnp.dot(x_ref[...], w1t_ref[...], preferred_element_type=jnp.float32)
    h = h + b1_ref[...]
    h = jnp.where(h > 0, h, 0.1 * h)
    o = jnp.dot(h, w2t_ref[...], preferred_element_type=jnp.float32)
    o = o + b2_ref[...]
    o_ref[0] = o[:, :_DH]
    o_ref[1] = o[:, _DH:]


def _sc_body(nchunk, h_hbm, rowi_hbm, coli_hbm, vali_hbm, out_hbm, y_hbm,
             rowv, colv, valv, rows_v, acc, isem, gsem, ssem0, ssem1):
    c = lax.axis_index("c")
    s = lax.axis_index("s")
    cbias = c * _NT
    epad = _NS * nchunk * _CH
    ebase = s * (nchunk * _CH)
    cbase = c * epad + ebase
    ngrp = nchunk * _GPC
    zerov = jnp.zeros((_L,), jnp.float32)

    def _zero_acc():
        def zbody(r, carry):
            for fb in range(_DH // _L):
                rows_v[r, pl.ds(fb * _L, _L)] = zerov
            return carry
        lax.fori_loop(0, _CH, zbody, 0)
        for off in range(0, _RPT, _CH):
            n = min(_CH, _RPT - off)
            pltpu.sync_copy(rows_v.at[pl.ds(0, n)],
                            acc.at[pl.ds(s * _RPT + off, n)])

        @pl.when(s == _NS - 1)
        def _():
            pltpu.sync_copy(rows_v.at[pl.ds(0, _NT - _NS * _RPT)],
                            acc.at[pl.ds(_NS * _RPT, _NT - _NS * _RPT)])

    def _pass(table_hbm, dst_fn):
        kper = _RGRP // _GPC

        def sca_desc(k, j):
            return pltpu.make_async_copy(
                rows_v.at[pl.ds(j * _G, _G)],
                acc.at[rowv.at[lax.rem(k, kper) * _GPC + j]],
                ssem0 if j == 0 else ssem1)

        pltpu.async_copy(coli_hbm.at[pl.ds(cbase, _CH)],
                         colv.at[pl.ds(0, _CH)], isem)

        def chunk(k, carry):
            e0 = ebase + k * _CH
            c0 = cbase + k * _CH
            co = lax.rem(k, 2) * _CH

            pltpu.make_async_copy(coli_hbm.at[pl.ds(c0, _CH)],
                                  colv.at[pl.ds(co, _CH)], isem).wait()

            gat = []
            for j in range(_GPC):
                @pl.when(k >= 1)
                def _(j=j):
                    sca_desc(k - 1, j).wait()
                gat.append(pltpu.async_copy(
                    table_hbm.at[colv.at[pl.ds(co + j * _G, _G)]],
                    rows_v.at[pl.ds(j * _G, _G)], gsem))

            @pl.when(lax.rem(k, kper) == 0)
            def _():
                g0 = s * ngrp + (k // kper) * _RGRP
                pltpu.sync_copy(rowi_hbm.at[pl.ds(g0, _RGRP)], rowv)

            @pl.when(k < nchunk - 1)
            def _():
                pltpu.async_copy(
                    coli_hbm.at[pl.ds(c0 + _CH, _CH)],
                    colv.at[pl.ds(_CH - co, _CH)], isem)

            pltpu.sync_copy(vali_hbm.at[pl.ds(e0, _CH)], valv)

            for j in range(_GPC):
                gat[j].wait()

                def scale(t, cc, j=j):
                    val16 = valv[pl.ds(j * _G + t * _L, _L)]
                    for u in range(_L):
                        r = j * _G + t * _L + u
                        v = val16[u]
                        for fb in range(_DH // _L):
                            rows_v[r, pl.ds(fb * _L, _L)] = (
                                rows_v[r, pl.ds(fb * _L, _L)] * v)
                    return cc
                lax.fori_loop(0, _G // _L, scale, 0)
                pltpu.async_copy(
                    rows_v.at[pl.ds(j * _G, _G)],
                    acc.at[rowv.at[lax.rem(k, kper) * _GPC + j]],
                    ssem0 if j == 0 else ssem1, add=True)
            return carry
        lax.fori_loop(0, nchunk, chunk, 0)
        sca_desc(nchunk - 1, 0).wait()
        sca_desc(nchunk - 1, 1).wait()
        plsc.subcore_barrier()
        dst_fn(s * _RPT, _RPT)

        @pl.when(s == _NS - 1)
        def _():
            dst_fn(_NS * _RPT, _NT - _NS * _RPT)

    def y_dst(r, n):
        pltpu.sync_copy(acc.at[pl.ds(r, n)],
                        y_hbm.at[pl.ds(cbias + r, n)])

    def out_dst(r, n):
        pltpu.sync_copy(acc.at[pl.ds(r, n)],
                        out_hbm.at[pl.ds(r, n), pl.ds(c * _DH, _DH)])

    _zero_acc()
    plsc.subcore_barrier()
    _pass(h_hbm, y_dst)
    _zero_acc()
    plsc.subcore_barrier()
    _pass(y_hbm, out_dst)


def kernel(seq_a, adj_indices, adj_values, node_emb, init_fea2, Wc, bc,
           W1, b1, W2, b2):
    del seq_a

    cm = pl.pallas_call(
        _codemap_body,
        out_shape=jax.ShapeDtypeStruct((init_fea2.shape[0], _D), jnp.float32),
    )(init_fea2, Wc.T, bc[None, :])
    x = jnp.concatenate([node_emb, cm], axis=0)

    nblk = _NT // _RB
    h_split = pl.pallas_call(
        _mlp_body,
        grid=(nblk,),
        in_specs=[
            pl.BlockSpec((_RB, _D), lambda i: (i, 0)),
            pl.BlockSpec((_D, W1.shape[0]), lambda i: (0, 0)),
            pl.BlockSpec((1, W1.shape[0]), lambda i: (0, 0)),
            pl.BlockSpec((W1.shape[0], _D), lambda i: (0, 0)),
            pl.BlockSpec((1, _D), lambda i: (0, 0)),
        ],
        out_specs=pl.BlockSpec((_NC, _RB, _DH), lambda i: (0, i, 0)),
        out_shape=jax.ShapeDtypeStruct((_NC, _NT, _DH), jnp.float32),
    )(x, W1.T, b1[None, :], W2.T, b2[None, :])
    h2 = h_split.reshape(_NC * _NT, _DH)

    e = adj_values.shape[0]
    epad = -(-e // (_NS * _CH)) * (_NS * _CH)
    rows = adj_indices[0].astype(jnp.int32)
    cols = adj_indices[1].astype(jnp.int32)
    vals = adj_values
    if epad != e:
        pad = epad - e
        rows = jnp.concatenate([rows, jnp.zeros((pad,), jnp.int32)])
        cols = jnp.concatenate([cols, jnp.zeros((pad,), jnp.int32)])
        vals = jnp.concatenate([vals, jnp.zeros((pad,), jnp.float32)])
    rows2 = rows.reshape(-1, _G)
    cols2 = jnp.concatenate([cols, cols + _NT])
    nchunk = epad // (_NS * _CH)

    mesh = plsc.VectorSubcoreMesh(core_axis_name="c", subcore_axis_name="s",
                                  num_cores=_NC, num_subcores=_NS)
    sc = pl.kernel(
        functools.partial(_sc_body, nchunk),
        out_type=(
            jax.ShapeDtypeStruct((_NT, _D), jnp.float32),
            jax.ShapeDtypeStruct((_NC * _NT, _DH), jnp.float32),
        ),
        mesh=mesh,
        scratch_types=[
            pltpu.VMEM((_RGRP, _G), jnp.int32),
            pltpu.VMEM((2 * _CH,), jnp.int32),
            pltpu.VMEM((_CH,), jnp.float32),
            pltpu.VMEM((_CH, _DH), jnp.float32),
            pltpu.VMEM_SHARED((_NT, _DH), jnp.float32),
            pltpu.SemaphoreType.DMA,
            pltpu.SemaphoreType.DMA,
            pltpu.SemaphoreType.DMA,
            pltpu.SemaphoreType.DMA,
        ],
    )
    out, _y = sc(h2, rows2, cols2, vals)
    return out


# speedup vs baseline: 1.1423x; 1.0151x over previous
import functools

import jax
import jax.numpy as jnp
from jax import lax
from jax.experimental import pallas as pl
from jax.experimental.pallas import tpu as pltpu
from jax.experimental.pallas import tpu_sc as plsc

_NT = 12048
_D = 256
_DH = 128
_NC = 2
_NS = 16
_L = 16
_G = 128
_GPC = 2
_CH = _G * _GPC
_REC = 2 * _CH
_RPT = 752
_RB = 2008


def _codemap_body(f2_ref, wct_ref, bc_ref, o_ref):
    o_ref[...] = (
        jnp.dot(f2_ref[...], wct_ref[...], preferred_element_type=jnp.float32)
        + bc_ref[...]
    )


def _mlp_body(x_ref, w1t_ref, b1_ref, w2t_ref, b2_ref, o_ref):
    h = jnp.dot(x_ref[...], w1t_ref[...], preferred_element_type=jnp.float32)
    h = h + b1_ref[...]
    h = jnp.where(h > 0, h, 0.1 * h)
    o = jnp.dot(h, w2t_ref[...], preferred_element_type=jnp.float32)
    o = o + b2_ref[...]
    o_ref[0] = o[:, :_DH]
    o_ref[1] = o[:, _DH:]


def _sc_body(nchunk, h_hbm, cvr_hbm, vali_hbm, out_hbm, y_hbm,
             cvr, valv, rows_v, acc, isem, gsem, ssem0, ssem1):
    c = lax.axis_index("c")
    s = lax.axis_index("s")
    cbias = c * _NT
    nchq = _NS * nchunk
    rbase = (c * nchq + s * nchunk) * _REC
    ebase = s * (nchunk * _CH)
    zerov = jnp.zeros((_L,), jnp.float32)

    def _zero_acc():
        def zbody(r, carry):
            for fb in range(_DH // _L):
                rows_v[r, pl.ds(fb * _L, _L)] = zerov
            return carry
        lax.fori_loop(0, _CH, zbody, 0)
        for off in range(0, _RPT, _CH):
            n = min(_CH, _RPT - off)
            pltpu.sync_copy(rows_v.at[pl.ds(0, n)],
                            acc.at[pl.ds(s * _RPT + off, n)])

        @pl.when(s == _NS - 1)
        def _():
            pltpu.sync_copy(rows_v.at[pl.ds(0, _NT - _NS * _RPT)],
                            acc.at[pl.ds(_NS * _RPT, _NT - _NS * _RPT)])

    def _pass(table_hbm, dst_fn):
        def sca_desc(k, j):
            co = lax.rem(k, 2) * _REC
            return pltpu.make_async_copy(
                rows_v.at[pl.ds(j * _G, _G)],
                acc.at[cvr.at[pl.ds(co + _CH + j * _G, _G)]],
                ssem0 if j == 0 else ssem1)

        pltpu.async_copy(cvr_hbm.at[pl.ds(rbase, _REC)],
                         cvr.at[pl.ds(0, _REC)], isem)
        pltpu.async_copy(vali_hbm.at[pl.ds(ebase, _CH)],
                         valv.at[pl.ds(0, _CH)], isem)

        def chunk(k, carry):
            r0 = rbase + k * _REC
            e0 = ebase + k * _CH
            co = lax.rem(k, 2) * _REC
            vo = lax.rem(k, 2) * _CH

            pltpu.make_async_copy(cvr_hbm.at[pl.ds(r0, _REC)],
                                  cvr.at[pl.ds(co, _REC)], isem).wait()
            pltpu.make_async_copy(vali_hbm.at[pl.ds(e0, _CH)],
                                  valv.at[pl.ds(vo, _CH)], isem).wait()

            gat = []
            for j in range(_GPC):
                @pl.when(k >= 1)
                def _(j=j):
                    sca_desc(k - 1, j).wait()
                gat.append(pltpu.async_copy(
                    table_hbm.at[cvr.at[pl.ds(co + j * _G, _G)]],
                    rows_v.at[pl.ds(j * _G, _G)], gsem))

            @pl.when(k < nchunk - 1)
            def _():
                pltpu.async_copy(
                    cvr_hbm.at[pl.ds(r0 + _REC, _REC)],
                    cvr.at[pl.ds(_REC - co, _REC)], isem)
                pltpu.async_copy(
                    vali_hbm.at[pl.ds(e0 + _CH, _CH)],
                    valv.at[pl.ds(_CH - vo, _CH)], isem)

            for j in range(_GPC):
                gat[j].wait()

                def scale(t, cc, j=j):
                    val16 = valv[pl.ds(vo + j * _G + t * _L, _L)]
                    for u in range(_L):
                        r = j * _G + t * _L + u
                        v = val16[u]
                        for fb in range(_DH // _L):
                            rows_v[r, pl.ds(fb * _L, _L)] = (
                                rows_v[r, pl.ds(fb * _L, _L)] * v)
                    return cc
                lax.fori_loop(0, _G // _L, scale, 0)
                pltpu.async_copy(
                    rows_v.at[pl.ds(j * _G, _G)],
                    acc.at[cvr.at[pl.ds(co + _CH + j * _G, _G)]],
                    ssem0 if j == 0 else ssem1, add=True)
            return carry
        lax.fori_loop(0, nchunk, chunk, 0)
        sca_desc(nchunk - 1, 0).wait()
        sca_desc(nchunk - 1, 1).wait()
        plsc.subcore_barrier()
        dst_fn(s * _RPT, _RPT)

        @pl.when(s == _NS - 1)
        def _():
            dst_fn(_NS * _RPT, _NT - _NS * _RPT)

    def y_dst(r, n):
        pltpu.sync_copy(acc.at[pl.ds(r, n)],
                        y_hbm.at[pl.ds(cbias + r, n)])

    def out_dst(r, n):
        pltpu.sync_copy(acc.at[pl.ds(r, n)],
                        out_hbm.at[pl.ds(r, n), pl.ds(c * _DH, _DH)])

    _zero_acc()
    plsc.subcore_barrier()
    _pass(h_hbm, y_dst)
    _zero_acc()
    plsc.subcore_barrier()
    _pass(y_hbm, out_dst)


def kernel(seq_a, adj_indices, adj_values, node_emb, init_fea2, Wc, bc,
           W1, b1, W2, b2):
    del seq_a

    cm = pl.pallas_call(
        _codemap_body,
        out_shape=jax.ShapeDtypeStruct((init_fea2.shape[0], _D), jnp.float32),
    )(init_fea2, Wc.T, bc[None, :])
    x = jnp.concatenate([node_emb, cm], axis=0)

    nblk = _NT // _RB
    h_split = pl.pallas_call(
        _mlp_body,
        grid=(nblk,),
        in_specs=[
            pl.BlockSpec((_RB, _D), lambda i: (i, 0)),
            pl.BlockSpec((_D, W1.shape[0]), lambda i: (0, 0)),
            pl.BlockSpec((1, W1.shape[0]), lambda i: (0, 0)),
            pl.BlockSpec((W1.shape[0], _D), lambda i: (0, 0)),
            pl.BlockSpec((1, _D), lambda i: (0, 0)),
        ],
        out_specs=pl.BlockSpec((_NC, _RB, _DH), lambda i: (0, i, 0)),
        out_shape=jax.ShapeDtypeStruct((_NC, _NT, _DH), jnp.float32),
    )(x, W1.T, b1[None, :], W2.T, b2[None, :])
    h2 = h_split.reshape(_NC * _NT, _DH)

    e = adj_values.shape[0]
    epad = -(-e // (_NS * _CH)) * (_NS * _CH)
    rows = adj_indices[0].astype(jnp.int32)
    cols = adj_indices[1].astype(jnp.int32)
    vals = adj_values
    if epad != e:
        pad = epad - e
        rows = jnp.concatenate([rows, jnp.zeros((pad,), jnp.int32)])
        cols = jnp.concatenate([cols, jnp.zeros((pad,), jnp.int32)])
        vals = jnp.concatenate([vals, jnp.zeros((pad,), jnp.float32)])
    nchunk = epad // (_NS * _CH)
    rr = rows.reshape(-1, _CH)
    cvr = jnp.concatenate([
        jnp.concatenate([(cols + c * _NT).reshape(-1, _CH), rr], axis=1)
        for c in range(_NC)], axis=0).reshape(-1)

    mesh = plsc.VectorSubcoreMesh(core_axis_name="c", subcore_axis_name="s",
                                  num_cores=_NC, num_subcores=_NS)
    sc = pl.kernel(
        functools.partial(_sc_body, nchunk),
        out_type=(
            jax.ShapeDtypeStruct((_NT, _D), jnp.float32),
            jax.ShapeDtypeStruct((_NC * _NT, _DH), jnp.float32),
        ),
        mesh=mesh,
        scratch_types=[
            pltpu.VMEM((2 * _REC,), jnp.int32),
            pltpu.VMEM((2 * _CH,), jnp.float32),
            pltpu.VMEM((_CH, _DH), jnp.float32),
            pltpu.VMEM_SHARED((_NT, _DH), jnp.float32),
            pltpu.SemaphoreType.DMA,
            pltpu.SemaphoreType.DMA,
            pltpu.SemaphoreType.DMA,
            pltpu.SemaphoreType.DMA,
        ],
    )
    out, _y = sc(h2, cvr, vals)
    return out
